# baseline (speedup 1.0000x reference)
import math

import jax
import jax.numpy as jnp
from jax import lax
from jax.experimental import pallas as pl
from jax.experimental.pallas import tpu as pltpu
from jax.experimental.pallas import tpu_sc as plsc

N = 10000
IN_FEATS = 128
NUM_HEADS = 8
OUT_FEATS = 16
FF = 4 * IN_FEATS
HW = 64
HPC = NUM_HEADS // 2
NP = 10240
NC = 2
NS = 16
C = 128
NCHUNK = 160
EPT = NCHUNK * C
EP = NS * EPT
ROWS_PER_TILE = NP // NS
INV_S = 1.0 / math.sqrt(NUM_HEADS * OUT_FEATS)
F32 = jnp.float32
I32 = jnp.int32


def _qkv_body(feat_ref, wq_ref, wk_ref, wv_ref, q_ref, k_ref, v_ref):
    f = feat_ref[...]
    dn = (((1,), (1,)), ((), ()))
    q = lax.dot_general(f, wq_ref[...], dn, preferred_element_type=F32) * INV_S
    k = lax.dot_general(f, wk_ref[...], dn, preferred_element_type=F32)
    v = lax.dot_general(f, wv_ref[...], dn, preferred_element_type=F32)
    q_ref[0] = q[:, :HW]
    q_ref[1] = q[:, HW:]
    k_ref[0] = k[:, :HW]
    k_ref[1] = k[:, HW:]
    v_ref[0] = v[:, :HW]
    v_ref[1] = v[:, HW:]


def _qkv(feat_p, Wq, Wk, Wv):
    BR = 1280
    bs_rows = pl.BlockSpec((BR, IN_FEATS), lambda i: (i, 0))
    bs_w = pl.BlockSpec((IN_FEATS, IN_FEATS), lambda i: (0, 0))
    bs_out = pl.BlockSpec((2, BR, HW), lambda i: (0, i, 0))
    return pl.pallas_call(
        _qkv_body,
        grid=(NP // BR,),
        in_specs=[bs_rows, bs_w, bs_w, bs_w],
        out_specs=[bs_out, bs_out, bs_out],
        out_shape=[jax.ShapeDtypeStruct((2, NP, HW), F32)] * 3,
    )(feat_p, Wq, Wk, Wv)


def _edge_body(idx_hbm, kt, qt, vt, macc_out, esum_out,
               idx0, idx1, sidx0, sidx1, krows0, qrows0, vrows0, eebuf0,
               krows1, qrows1, vrows1, eebuf1, ebuf_t, zbuf, zeb,
               macc_sh, esum_sh, gsem0, gsem1, ssem0, ssem1, isem0, isem1):
    c = lax.axis_index("c")
    s = lax.axis_index("s")
    row_base = s * NCHUNK

    pltpu.sync_copy(idx_hbm.at[c, row_base], idx0)
    pltpu.async_copy(kt.at[idx0.at[0]], krows0, gsem0)
    pltpu.async_copy(qt.at[idx0.at[1]], qrows0, gsem0)
    pltpu.async_copy(vt.at[idx0.at[0]], vrows0, gsem0)
    pltpu.async_copy(idx_hbm.at[c, row_base + 1], idx1, isem1)

    def zrow(r, x):
        for j in range(HW // 16):
            zbuf[r, pl.ds(j * 16, 16)] = jnp.zeros((16,), F32)
        return x
    lax.fori_loop(0, 64, zrow, 0)

    def zrow2(r, x):
        zeb[r, :] = jnp.zeros((16,), F32)
        return x
    lax.fori_loop(0, 64, zrow2, 0)

    def zrow3(r, x):
        eebuf0[r, :] = jnp.zeros((16,), F32)
        eebuf1[r, :] = jnp.zeros((16,), F32)
        return x
    lax.fori_loop(0, C, zrow3, 0)

    r0 = s * ROWS_PER_TILE
    for i in range(ROWS_PER_TILE // 64):
        pltpu.sync_copy(zbuf, macc_sh.at[pl.ds(r0 + i * 64, 64)])
        pltpu.sync_copy(zeb, esum_sh.at[pl.ds(r0 + i * 64, 64)])
    plsc.subcore_barrier()

    iota16 = lax.iota(I32, 16)
    sets = ((idx0, sidx0, krows0, qrows0, vrows0, eebuf0, gsem0, ssem0, isem0),
            (idx1, sidx1, krows1, qrows1, vrows1, eebuf1, gsem1, ssem1, isem1))

    def issue_idx(i_chunk, bufset):
        idx = bufset[0]
        isem = bufset[8]
        pltpu.async_copy(idx_hbm.at[c, row_base + i_chunk], idx, isem)

    def drain_idx(bufset):
        idx = bufset[0]
        isem = bufset[8]
        pltpu.make_async_copy(idx_hbm.at[c, row_base], idx, isem).wait()

    def issue_gathers(bufset):
        idx, _, krows, qrows, vrows = bufset[:5]
        gsem = bufset[6]
        pltpu.async_copy(kt.at[idx.at[0]], krows, gsem)
        pltpu.async_copy(qt.at[idx.at[1]], qrows, gsem)
        pltpu.async_copy(vt.at[idx.at[0]], vrows, gsem)

    def drain_gathers(bufset):
        idx, _, krows, qrows, vrows = bufset[:5]
        gsem = bufset[6]
        pltpu.make_async_copy(kt.at[idx.at[0]], krows, gsem).wait()
        pltpu.make_async_copy(qt.at[idx.at[1]], qrows, gsem).wait()
        pltpu.make_async_copy(vt.at[idx.at[0]], vrows, gsem).wait()

    lane15 = iota16 == 15

    def save_scatter_idx(bufset):
        idx, sidx = bufset[:2]
        for t in range(128 // 16):
            sidx[0, pl.ds(t * 16, 16)] = idx[2, pl.ds(t * 16, 16)]

    def compute_chunk(bufset):
        idx, sidx, krows, qrows, vrows, eebuf = bufset[:6]

        lane15v = jnp.full((16,), 15, I32)

        def group(g, y):
            for cc in range(16):
                for h in range(HPC):
                    sl = pl.ds(h * OUT_FEATS, OUT_FEATS)
                    kv = krows[g * 16 + cc, sl]
                    qv = qrows[g * 16 + cc, sl]
                    cum = plsc.cumsum(kv * qv)
                    tot = cum.at[lane15v].get(mode='promise_in_bounds')
                    ee = jnp.exp(tot)
                    plsc.store_scatter(
                        eebuf, [jnp.full((16,), g * 16 + cc, I32),
                                jnp.full((16,), h, I32)], ee, mask=lane15)
                    vrows[g * 16 + cc, sl] = vrows[g * 16 + cc, sl] * ee
            return y
        lax.fori_loop(0, C // 16, group, 0)

    def issue_scatters(bufset):
        sidx, _, _, vrows, eebuf = bufset[1:6]
        ssem = bufset[7]
        pltpu.async_copy(eebuf, esum_sh.at[sidx.at[0]], ssem, add=True)
        pltpu.async_copy(vrows, macc_sh.at[sidx.at[0]], ssem, add=True)

    def drain_scatters(bufset):
        sidx, _, _, vrows, eebuf = bufset[1:6]
        ssem = bufset[7]
        pltpu.make_async_copy(eebuf, esum_sh.at[sidx.at[0]], ssem).wait()
        pltpu.make_async_copy(vrows, macc_sh.at[sidx.at[0]], ssem).wait()

    def pair_body(j, x):
        for b in range(2):
            i_chunk = 2 * j + b
            cur = sets[b]
            nxt = sets[1 - b]
            drain_gathers(cur)
            save_scatter_idx(cur)

            @pl.when(j < NCHUNK // 2 - 1)
            def _():
                issue_idx(i_chunk + 2, cur)
            if b == 0:
                @pl.when(j > 0)
                def _():
                    drain_scatters(nxt)
                drain_idx(nxt)
                issue_gathers(nxt)
            else:
                drain_scatters(nxt)

                @pl.when(j < NCHUNK // 2 - 1)
                def _():
                    drain_idx(nxt)
                    issue_gathers(nxt)
            compute_chunk(cur)
            issue_scatters(cur)
        return x

    lax.fori_loop(0, NCHUNK // 2, pair_body, 0)
    drain_scatters(sets[1])
    plsc.subcore_barrier()

    pltpu.sync_copy(macc_sh.at[pl.ds(r0, ROWS_PER_TILE)],
                    macc_out.at[pl.ds(c * NP + r0, ROWS_PER_TILE)])
    pltpu.sync_copy(esum_sh.at[pl.ds(r0, ROWS_PER_TILE)],
                    esum_out.at[pl.ds(c * NP + r0, ROWS_PER_TILE)])


_edge_call = pl.kernel(
    _edge_body,
    out_type=(jax.ShapeDtypeStruct((2 * NP, HW), F32),
              jax.ShapeDtypeStruct((2 * NP, 16), F32)),
    mesh=plsc.VectorSubcoreMesh(core_axis_name="c", subcore_axis_name="s"),
    scratch_types=[
        pltpu.VMEM((3, 128), I32),
        pltpu.VMEM((3, 128), I32),
        pltpu.VMEM((1, 128), I32),
        pltpu.VMEM((1, 128), I32),
        pltpu.VMEM((C, HW), F32),
        pltpu.VMEM((C, HW), F32),
        pltpu.VMEM((C, HW), F32),
        pltpu.VMEM((C, 16), F32),
        pltpu.VMEM((C, HW), F32),
        pltpu.VMEM((C, HW), F32),
        pltpu.VMEM((C, HW), F32),
        pltpu.VMEM((C, 16), F32),
        pltpu.VMEM((HPC * 16,), F32),
        pltpu.VMEM((64, HW), F32),
        pltpu.VMEM((64, 16), F32),
        pltpu.VMEM_SHARED((NP, HW), F32),
        pltpu.VMEM_SHARED((NP, 16), F32),
        pltpu.SemaphoreType.DMA,
        pltpu.SemaphoreType.DMA,
        pltpu.SemaphoreType.DMA,
        pltpu.SemaphoreType.DMA,
        pltpu.SemaphoreType.DMA,
        pltpu.SemaphoreType.DMA,
    ],
    compiler_params=pltpu.CompilerParams(needs_layout_passes=False,
                                         use_tc_tiling_on_sc=False),
)


def _post_body(m0_ref, m1_ref, e0, e1, feat_ref, g_ref, b_ref, w1_ref, b1_ref,
               al_ref, w2_ref, b2_ref, out_ref):
    ft = jnp.concatenate([m0_ref[...], m1_ref[...]], axis=1)
    j16 = lax.broadcasted_iota(I32, (16, IN_FEATS), 0)
    f16 = lax.broadcasted_iota(I32, (16, IN_FEATS), 1) // OUT_FEATS
    m0 = (j16 == f16).astype(F32)
    m1 = (j16 + 4 == f16).astype(F32)
    dn0 = (((1,), (0,)), ((), ()))
    esb = (lax.dot_general(e0[...], m0, dn0, preferred_element_type=F32) +
           lax.dot_general(e1[...], m1, dn0, preferred_element_type=F32))
    esb = jnp.where(esb == 0.0, 1.0, esb)
    rst = ft / esb + feat_ref[...]
    g = g_ref[...]
    b = b_ref[...]
    mu = jnp.mean(rst, axis=-1, keepdims=True)
    var = jnp.mean((rst - mu) ** 2, axis=-1, keepdims=True)
    rst = (rst - mu) * lax.rsqrt(var + 1e-5) * g + b
    dn = (((1,), (1,)), ((), ()))
    h = lax.dot_general(rst, w1_ref[...], dn, preferred_element_type=F32) + b1_ref[...]
    h = jnp.maximum(h, 0.0) + al_ref[...] * jnp.minimum(h, 0.0)
    h = lax.dot_general(h, w2_ref[...], dn, preferred_element_type=F32) + b2_ref[...]
    x = rst + h
    mu2 = jnp.mean(x, axis=-1, keepdims=True)
    var2 = jnp.mean((x - mu2) ** 2, axis=-1, keepdims=True)
    out_ref[...] = (x - mu2) * lax.rsqrt(var2 + 1e-5) * g + b


def _post(m0, m1, e0, e1, feat_p, ln_g, ln_b, W1, b1, alpha, W2, b2):
    BR = 1280
    bs_rows = pl.BlockSpec((BR, IN_FEATS), lambda i: (i, 0))
    bs_m = pl.BlockSpec((BR, HW), lambda i: (i, 0))
    bs_es = pl.BlockSpec((BR, 16), lambda i: (i, 0))
    full = lambda shape: pl.BlockSpec(shape, lambda i: (0, 0))
    return pl.pallas_call(
        _post_body,
        grid=(NP // BR,),
        in_specs=[bs_m, bs_m, bs_es, bs_es, bs_rows,
                  full((1, IN_FEATS)), full((1, IN_FEATS)),
                  full((FF, IN_FEATS)), full((1, FF)), full((1, FF)),
                  full((IN_FEATS, FF)), full((1, IN_FEATS))],
        out_specs=bs_rows,
        out_shape=jax.ShapeDtypeStruct((NP, IN_FEATS), F32),
    )(m0, m1, e0, e1, feat_p, ln_g, ln_b, W1, b1, alpha, W2, b2)


@jax.jit
def kernel(feat, edge_index, Wq, Wk, Wv, ln_g, ln_b, W1, b1, alpha, W2, b2):
    feat_p = jnp.zeros((NP, IN_FEATS), F32).at[:N].set(feat)
    q, k, v = _qkv(feat_p, Wq, Wk, Wv)
    e = edge_index.shape[1]
    pad = EP - e
    src = jnp.concatenate([edge_index[0].astype(I32), jnp.full((pad,), N, I32)])
    dst = jnp.concatenate([edge_index[1].astype(I32), jnp.full((pad,), N, I32)])
    src = src.reshape(EP // 128, 128)
    dst = dst.reshape(EP // 128, 128)
    offs = (jnp.arange(2, dtype=I32) * NP)[:, None, None]
    idx_all = jnp.stack([
        jnp.broadcast_to(src, (2,) + src.shape) + offs,
        jnp.broadcast_to(dst, (2,) + dst.shape) + offs,
        jnp.broadcast_to(dst, (2,) + dst.shape),
    ], axis=2)
    macc, esum = _edge_call(idx_all,
                            k.reshape(2 * NP, HW), q.reshape(2 * NP, HW),
                            v.reshape(2 * NP, HW))
    out = _post(macc[:NP], macc[NP:], esum[:NP], esum[NP:], feat_p,
                ln_g.reshape(1, IN_FEATS), ln_b.reshape(1, IN_FEATS),
                W1, b1.reshape(1, FF), alpha.reshape(1, FF),
                W2, b2.reshape(1, IN_FEATS))
    return out[:N]


# speedup vs baseline: 1.4966x; 1.4966x over previous
import math

import jax
import jax.numpy as jnp
from jax import lax
from jax.experimental import pallas as pl
from jax.experimental.pallas import tpu as pltpu
from jax.experimental.pallas import tpu_sc as plsc

N = 10000
IN_FEATS = 128
NUM_HEADS = 8
OUT_FEATS = 16
FF = 4 * IN_FEATS
HW = 64
HPC = NUM_HEADS // 2
NP = 10240
NC = 2
NS = 16
C = 128
NCHUNK = 160
EPT = NCHUNK * C
EP = NS * EPT
ROWS_PER_TILE = NP // NS
INV_S = 1.0 / math.sqrt(NUM_HEADS * OUT_FEATS)
F32 = jnp.float32
I32 = jnp.int32


def _qkv_body(feat_ref, wq_ref, wk_ref, wv_ref, q_ref, k_ref, v_ref):
    f = feat_ref[...]
    dn = (((1,), (1,)), ((), ()))
    q = lax.dot_general(f, wq_ref[...], dn, preferred_element_type=F32) * INV_S
    k = lax.dot_general(f, wk_ref[...], dn, preferred_element_type=F32)
    v = lax.dot_general(f, wv_ref[...], dn, preferred_element_type=F32)
    q_ref[0] = q[:, :HW]
    q_ref[1] = q[:, HW:]
    k_ref[0] = k[:, :HW]
    k_ref[1] = k[:, HW:]
    v_ref[0] = v[:, :HW]
    v_ref[1] = v[:, HW:]


def _qkv(feat_p, Wq, Wk, Wv):
    BR = 1280
    bs_rows = pl.BlockSpec((BR, IN_FEATS), lambda i: (i, 0))
    bs_w = pl.BlockSpec((IN_FEATS, IN_FEATS), lambda i: (0, 0))
    bs_out = pl.BlockSpec((2, BR, HW), lambda i: (0, i, 0))
    return pl.pallas_call(
        _qkv_body,
        grid=(NP // BR,),
        in_specs=[bs_rows, bs_w, bs_w, bs_w],
        out_specs=[bs_out, bs_out, bs_out],
        out_shape=[jax.ShapeDtypeStruct((2, NP, HW), F32)] * 3,
    )(feat_p, Wq, Wk, Wv)


def _edge_body(idx_hbm, kt, qt, vt, macc_out, esum_out,
               idx0, idx1, sidx0, sidx1, krows0, qrows0, vrows0, eebuf0,
               krows1, qrows1, vrows1, eebuf1, ebuf_t, zbuf, zeb,
               macc_sh, esum_sh, gsem0, gsem1, ssem0, ssem1, isem0, isem1):
    c = lax.axis_index("c")
    s = lax.axis_index("s")
    row_base = s * NCHUNK

    pltpu.sync_copy(idx_hbm.at[c, row_base], idx0)
    pltpu.async_copy(kt.at[idx0.at[0]], krows0, gsem0)
    pltpu.async_copy(qt.at[idx0.at[1]], qrows0, gsem0)
    pltpu.async_copy(vt.at[idx0.at[0]], vrows0, gsem0)
    pltpu.async_copy(idx_hbm.at[c, row_base + 1], idx1, isem1)

    def zrow(r, x):
        for j in range(HW // 16):
            zbuf[r, pl.ds(j * 16, 16)] = jnp.zeros((16,), F32)
        return x
    lax.fori_loop(0, 64, zrow, 0)

    def zrow2(r, x):
        zeb[r, :] = jnp.zeros((16,), F32)
        return x
    lax.fori_loop(0, 64, zrow2, 0)

    def zrow3(r, x):
        eebuf0[r, :] = jnp.zeros((16,), F32)
        eebuf1[r, :] = jnp.zeros((16,), F32)
        return x
    lax.fori_loop(0, C, zrow3, 0)

    r0 = s * ROWS_PER_TILE
    for i in range(ROWS_PER_TILE // 64):
        pltpu.sync_copy(zbuf, macc_sh.at[pl.ds(r0 + i * 64, 64)])
        pltpu.sync_copy(zeb, esum_sh.at[pl.ds(r0 + i * 64, 64)])
    plsc.subcore_barrier()

    iota16 = lax.iota(I32, 16)
    sets = ((idx0, sidx0, krows0, qrows0, vrows0, eebuf0, gsem0, ssem0, isem0),
            (idx1, sidx1, krows1, qrows1, vrows1, eebuf1, gsem1, ssem1, isem1))

    def issue_idx(i_chunk, bufset):
        idx = bufset[0]
        isem = bufset[8]
        pltpu.async_copy(idx_hbm.at[c, row_base + i_chunk], idx, isem)

    def drain_idx(bufset):
        idx = bufset[0]
        isem = bufset[8]
        pltpu.make_async_copy(idx_hbm.at[c, row_base], idx, isem).wait()

    def issue_gathers(bufset):
        idx, _, krows, qrows, vrows = bufset[:5]
        gsem = bufset[6]
        pltpu.async_copy(kt.at[idx.at[0]], krows, gsem)
        pltpu.async_copy(qt.at[idx.at[1]], qrows, gsem)
        pltpu.async_copy(vt.at[idx.at[0]], vrows, gsem)

    def drain_gathers(bufset):
        idx, _, krows, qrows, vrows = bufset[:5]
        gsem = bufset[6]
        pltpu.make_async_copy(kt.at[idx.at[0]], krows, gsem).wait()
        pltpu.make_async_copy(qt.at[idx.at[1]], qrows, gsem).wait()
        pltpu.make_async_copy(vt.at[idx.at[0]], vrows, gsem).wait()

    lane15 = iota16 == 15

    def save_scatter_idx(bufset):
        idx, sidx = bufset[:2]
        for t in range(128 // 16):
            sidx[0, pl.ds(t * 16, 16)] = idx[2, pl.ds(t * 16, 16)]

    def compute_chunk(bufset):
        idx, sidx, krows, qrows, vrows, eebuf = bufset[:6]

        def group(g, y):
            rowids = g * 16 + iota16
            for cc in range(16):
                for h in range(HPC):
                    sl = pl.ds(h * OUT_FEATS, OUT_FEATS)
                    kv = krows[g * 16 + cc, sl]
                    qv = qrows[g * 16 + cc, sl]
                    cum = plsc.cumsum(kv * qv)
                    plsc.store_scatter(ebuf_t, [jnp.full((16,), h * 16 + cc, I32)],
                                       cum, mask=lane15)
            for h in range(HPC):
                ev = ebuf_t[pl.ds(h * 16, 16)]
                eeh = jnp.exp(ev)
                plsc.store_scatter(eebuf, [rowids, jnp.full((16,), h, I32)], eeh)
                for cc in range(16):
                    bc = eeh.at[jnp.full((16,), cc, I32)].get(
                        mode='promise_in_bounds')
                    sl = pl.ds(h * OUT_FEATS, OUT_FEATS)
                    vrows[g * 16 + cc, sl] = vrows[g * 16 + cc, sl] * bc
            return y
        lax.fori_loop(0, C // 16, group, 0)

    def issue_scatters(bufset):
        sidx, _, _, vrows, eebuf = bufset[1:6]
        ssem = bufset[7]
        pltpu.async_copy(eebuf, esum_sh.at[sidx.at[0]], ssem, add=True)
        pltpu.async_copy(vrows, macc_sh.at[sidx.at[0]], ssem, add=True)

    def drain_scatters(bufset):
        sidx, _, _, vrows, eebuf = bufset[1:6]
        ssem = bufset[7]
        pltpu.make_async_copy(eebuf, esum_sh.at[sidx.at[0]], ssem).wait()
        pltpu.make_async_copy(vrows, macc_sh.at[sidx.at[0]], ssem).wait()

    def pair_body(j, x):
        for b in range(2):
            i_chunk = 2 * j + b
            cur = sets[b]
            nxt = sets[1 - b]
            drain_gathers(cur)
            save_scatter_idx(cur)

            @pl.when(j < NCHUNK // 2 - 1)
            def _():
                issue_idx(i_chunk + 2, cur)
            if b == 0:
                @pl.when(j > 0)
                def _():
                    drain_scatters(nxt)
                drain_idx(nxt)
                issue_gathers(nxt)
            else:
                drain_scatters(nxt)

                @pl.when(j < NCHUNK // 2 - 1)
                def _():
                    drain_idx(nxt)
                    issue_gathers(nxt)
            compute_chunk(cur)
            issue_scatters(cur)
        return x

    lax.fori_loop(0, NCHUNK // 2, pair_body, 0)
    drain_scatters(sets[1])
    plsc.subcore_barrier()

    pltpu.sync_copy(macc_sh.at[pl.ds(r0, ROWS_PER_TILE)],
                    macc_out.at[pl.ds(c * NP + r0, ROWS_PER_TILE)])
    pltpu.sync_copy(esum_sh.at[pl.ds(r0, ROWS_PER_TILE)],
                    esum_out.at[pl.ds(c * NP + r0, ROWS_PER_TILE)])


_edge_call = pl.kernel(
    _edge_body,
    out_type=(jax.ShapeDtypeStruct((2 * NP, HW), F32),
              jax.ShapeDtypeStruct((2 * NP, 16), F32)),
    mesh=plsc.VectorSubcoreMesh(core_axis_name="c", subcore_axis_name="s"),
    scratch_types=[
        pltpu.VMEM((3, 128), I32),
        pltpu.VMEM((3, 128), I32),
        pltpu.VMEM((1, 128), I32),
        pltpu.VMEM((1, 128), I32),
        pltpu.VMEM((C, HW), F32),
        pltpu.VMEM((C, HW), F32),
        pltpu.VMEM((C, HW), F32),
        pltpu.VMEM((C, 16), F32),
        pltpu.VMEM((C, HW), F32),
        pltpu.VMEM((C, HW), F32),
        pltpu.VMEM((C, HW), F32),
        pltpu.VMEM((C, 16), F32),
        pltpu.VMEM((HPC * 16,), F32),
        pltpu.VMEM((64, HW), F32),
        pltpu.VMEM((64, 16), F32),
        pltpu.VMEM_SHARED((NP, HW), F32),
        pltpu.VMEM_SHARED((NP, 16), F32),
        pltpu.SemaphoreType.DMA,
        pltpu.SemaphoreType.DMA,
        pltpu.SemaphoreType.DMA,
        pltpu.SemaphoreType.DMA,
        pltpu.SemaphoreType.DMA,
        pltpu.SemaphoreType.DMA,
    ],
    compiler_params=pltpu.CompilerParams(needs_layout_passes=False,
                                         use_tc_tiling_on_sc=False),
)


def _post_body(m0_ref, m1_ref, e0, e1, feat_ref, g_ref, b_ref, w1_ref, b1_ref,
               al_ref, w2_ref, b2_ref, out_ref):
    ft = jnp.concatenate([m0_ref[...], m1_ref[...]], axis=1)
    j16 = lax.broadcasted_iota(I32, (16, IN_FEATS), 0)
    f16 = lax.broadcasted_iota(I32, (16, IN_FEATS), 1) // OUT_FEATS
    m0 = (j16 == f16).astype(F32)
    m1 = (j16 + 4 == f16).astype(F32)
    dn0 = (((1,), (0,)), ((), ()))
    esb = (lax.dot_general(e0[...], m0, dn0, preferred_element_type=F32) +
           lax.dot_general(e1[...], m1, dn0, preferred_element_type=F32))
    esb = jnp.where(esb == 0.0, 1.0, esb)
    rst = ft / esb + feat_ref[...]
    g = g_ref[...]
    b = b_ref[...]
    mu = jnp.mean(rst, axis=-1, keepdims=True)
    var = jnp.mean((rst - mu) ** 2, axis=-1, keepdims=True)
    rst = (rst - mu) * lax.rsqrt(var + 1e-5) * g + b
    dn = (((1,), (1,)), ((), ()))
    h = lax.dot_general(rst, w1_ref[...], dn, preferred_element_type=F32) + b1_ref[...]
    h = jnp.maximum(h, 0.0) + al_ref[...] * jnp.minimum(h, 0.0)
    h = lax.dot_general(h, w2_ref[...], dn, preferred_element_type=F32) + b2_ref[...]
    x = rst + h
    mu2 = jnp.mean(x, axis=-1, keepdims=True)
    var2 = jnp.mean((x - mu2) ** 2, axis=-1, keepdims=True)
    out_ref[...] = (x - mu2) * lax.rsqrt(var2 + 1e-5) * g + b


def _post(m0, m1, e0, e1, feat_p, ln_g, ln_b, W1, b1, alpha, W2, b2):
    BR = 1280
    bs_rows = pl.BlockSpec((BR, IN_FEATS), lambda i: (i, 0))
    bs_m = pl.BlockSpec((BR, HW), lambda i: (i, 0))
    bs_es = pl.BlockSpec((BR, 16), lambda i: (i, 0))
    full = lambda shape: pl.BlockSpec(shape, lambda i: (0, 0))
    return pl.pallas_call(
        _post_body,
        grid=(NP // BR,),
        in_specs=[bs_m, bs_m, bs_es, bs_es, bs_rows,
                  full((1, IN_FEATS)), full((1, IN_FEATS)),
                  full((FF, IN_FEATS)), full((1, FF)), full((1, FF)),
                  full((IN_FEATS, FF)), full((1, IN_FEATS))],
        out_specs=bs_rows,
        out_shape=jax.ShapeDtypeStruct((NP, IN_FEATS), F32),
    )(m0, m1, e0, e1, feat_p, ln_g, ln_b, W1, b1, alpha, W2, b2)


@jax.jit
def kernel(feat, edge_index, Wq, Wk, Wv, ln_g, ln_b, W1, b1, alpha, W2, b2):
    feat_p = jnp.zeros((NP, IN_FEATS), F32).at[:N].set(feat)
    q, k, v = _qkv(feat_p, Wq, Wk, Wv)
    e = edge_index.shape[1]
    pad = EP - e
    src = jnp.concatenate([edge_index[0].astype(I32), jnp.full((pad,), N, I32)])
    dst = jnp.concatenate([edge_index[1].astype(I32), jnp.full((pad,), N, I32)])
    src = src.reshape(EP // 128, 128)
    dst = dst.reshape(EP // 128, 128)
    offs = (jnp.arange(2, dtype=I32) * NP)[:, None, None]
    idx_all = jnp.stack([
        jnp.broadcast_to(src, (2,) + src.shape) + offs,
        jnp.broadcast_to(dst, (2,) + dst.shape) + offs,
        jnp.broadcast_to(dst, (2,) + dst.shape),
    ], axis=2)
    macc, esum = _edge_call(idx_all,
                            k.reshape(2 * NP, HW), q.reshape(2 * NP, HW),
                            v.reshape(2 * NP, HW))
    out = _post(macc[:NP], macc[NP:], esum[:NP], esum[NP:], feat_p,
                ln_g.reshape(1, IN_FEATS), ln_b.reshape(1, IN_FEATS),
                W1, b1.reshape(1, FF), alpha.reshape(1, FF),
                W2, b2.reshape(1, IN_FEATS))
    return out[:N]
